# baseline (speedup 1.0000x reference)
import functools

import jax
import jax.numpy as jnp
from jax import lax
from jax.experimental import pallas as pl
from jax.experimental.pallas import tpu as pltpu
from jax.experimental.pallas import tpu_sc as plsc

NUM_CLASSES = 10000
FEATURE_DIM = 512
BATCH = 4096

NC = 2
NS = 16
L = 16
NW = NC * NS
ROWS_PER_W = BATCH // NW
CHUNK = 32
CHUNK_SIZES = (16, 16, 32, 32, 32)
CHUNK_OFFS = (0, 16, 32, 64, 96)
NCHUNK = len(CHUNK_SIZES)
SLICES = FEATURE_DIM // L
NBUF = 3


def _body(feat_hbm, labels_hbm, cent_hbm, out_hbm,
          idx_v, fb0, fb1, fb2, cb0, cb1, cb2, res_v,
          sem_f0, sem_f1, sem_f2, sem_c0, sem_c1, sem_c2):
  wid = lax.axis_index("s") * NC + lax.axis_index("c")
  base = wid * ROWS_PER_W

  fbufs = (fb0, fb1, fb2)
  cbufs = (cb0, cb1, cb2)
  fsems = (sem_f0, sem_f1, sem_f2)
  csems = (sem_c0, sem_c1, sem_c2)

  def start_feat(g):
    s = g % NBUF
    n = CHUNK_SIZES[g]
    return pltpu.async_copy(
        feat_hbm.at[pl.ds(base + CHUNK_OFFS[g], n), :],
        fbufs[s].at[pl.ds(0, n), :], fsems[s])

  def start_cent(g):
    s = g % NBUF
    n = CHUNK_SIZES[g]
    return pltpu.async_copy(
        cent_hbm.at[idx_v.at[pl.ds(CHUNK_OFFS[g], n)]],
        cbufs[s].at[pl.ds(0, n), :], csems[s])

  def start(g):
    return start_feat(g), start_cent(g)

  NA = 4

  def accumulate(fb, cb, nrows, accs):
    def row_body(r, a):
      a = list(a)
      for j in range(SLICES):
        f = fb[r, pl.ds(j * L, L)]
        c = cb[r, pl.ds(j * L, L)]
        d = f - c
        a[j % NA] = a[j % NA] + d * d
      return tuple(a)
    return plsc.parallel_loop(0, nrows, unroll=2, carry=accs)(row_body)

  accs = tuple(jnp.zeros((L,), jnp.float32) for _ in range(NA))
  handles = {0: (start_feat(0), None)}
  pltpu.sync_copy(labels_hbm.at[pl.ds(base, 16)], idx_v.at[pl.ds(0, 16)])
  handles[0] = (handles[0][0], start_cent(0))
  pltpu.sync_copy(labels_hbm.at[pl.ds(base + 16, ROWS_PER_W - 16)],
                  idx_v.at[pl.ds(16, ROWS_PER_W - 16)])

  def ensure(k):
    if 0 <= k < NCHUNK and k not in handles:
      handles[k] = start(k)

  ensure(1)
  for g in range(NCHUNK):
    ensure(g + 2)
    handles[g][0].wait()
    handles[g][1].wait()
    accs = accumulate(fbufs[g % NBUF], cbufs[g % NBUF], CHUNK_SIZES[g], accs)

  acc = accs[0]
  for a in accs[1:]:
    acc = acc + a
  res_v[...] = acc
  pltpu.sync_copy(res_v, out_hbm.at[wid])


@jax.jit
def _center_loss(features, labels, centers):
  labels2 = labels.astype(jnp.int32)
  mesh = plsc.VectorSubcoreMesh(core_axis_name="c", subcore_axis_name="s")
  run = pl.kernel(
      _body,
      out_type=jax.ShapeDtypeStruct((NW, L), jnp.float32),
      mesh=mesh,
      scratch_types=[
          pltpu.VMEM((ROWS_PER_W,), jnp.int32),
          pltpu.VMEM((CHUNK, FEATURE_DIM), jnp.float32),
          pltpu.VMEM((CHUNK, FEATURE_DIM), jnp.float32),
          pltpu.VMEM((CHUNK, FEATURE_DIM), jnp.float32),
          pltpu.VMEM((CHUNK, FEATURE_DIM), jnp.float32),
          pltpu.VMEM((CHUNK, FEATURE_DIM), jnp.float32),
          pltpu.VMEM((CHUNK, FEATURE_DIM), jnp.float32),
          pltpu.VMEM((L,), jnp.float32),
          pltpu.SemaphoreType.DMA,
          pltpu.SemaphoreType.DMA,
          pltpu.SemaphoreType.DMA,
          pltpu.SemaphoreType.DMA,
          pltpu.SemaphoreType.DMA,
          pltpu.SemaphoreType.DMA,
      ],
  )
  partials = run(features, labels2, centers)
  return jnp.sum(partials) / jnp.float32(BATCH * FEATURE_DIM)


def kernel(features, labels, centers):
  return _center_loss(features, labels, centers)


# speedup vs baseline: 1.0014x; 1.0014x over previous
import jax
import jax.numpy as jnp
from jax import lax
from jax.experimental import pallas as pl
from jax.experimental.pallas import tpu as pltpu
from jax.experimental.pallas import tpu_sc as plsc

NUM_CLASSES = 10000
FEATURE_DIM = 512
BATCH = 4096

NC = 2
NS = 16
L = 16
NW = NC * NS
ROWS_PER_W = BATCH // NW
CHUNK = 32
CHUNK_SIZES = (16, 16, 32, 32, 32)
CHUNK_OFFS = (0, 16, 32, 64, 96)
NCHUNK = len(CHUNK_SIZES)
SLICES = FEATURE_DIM // L
NBUF = 3


def _body(feat_hbm, labels_hbm, cent_hbm, out_hbm,
          idx_v, fb0, fb1, fb2, cb0, cb1, cb2, res_v,
          sem_f0, sem_f1, sem_f2, sem_c0, sem_c1, sem_c2):
  wid = lax.axis_index("s") * NC + lax.axis_index("c")
  base = wid * ROWS_PER_W

  fbufs = (fb0, fb1, fb2)
  cbufs = (cb0, cb1, cb2)
  fsems = (sem_f0, sem_f1, sem_f2)
  csems = (sem_c0, sem_c1, sem_c2)

  def start_feat(g):
    s = g % NBUF
    n = CHUNK_SIZES[g]
    return pltpu.async_copy(
        feat_hbm.at[pl.ds(base + CHUNK_OFFS[g], n), :],
        fbufs[s].at[pl.ds(0, n), :], fsems[s])

  def start_cent(g):
    s = g % NBUF
    n = CHUNK_SIZES[g]
    return pltpu.async_copy(
        cent_hbm.at[idx_v.at[pl.ds(CHUNK_OFFS[g], n)]],
        cbufs[s].at[pl.ds(0, n), :], csems[s])

  def start(g):
    return start_feat(g), start_cent(g)

  NA = 4

  def accumulate(fb, cb, nrows, accs):
    def row_body(r, a):
      a = list(a)
      for j in range(SLICES):
        f = fb[r, pl.ds(j * L, L)]
        c = cb[r, pl.ds(j * L, L)]
        d = f - c
        a[j % NA] = a[j % NA] + d * d
      return tuple(a)
    return plsc.parallel_loop(0, nrows, unroll=2, carry=accs)(row_body)

  accs = tuple(jnp.zeros((L,), jnp.float32) for _ in range(NA))
  handles = {0: (start_feat(0), None)}
  pltpu.sync_copy(labels_hbm.at[pl.ds(base, 16)], idx_v.at[pl.ds(0, 16)])
  handles[0] = (handles[0][0], start_cent(0))
  pltpu.sync_copy(labels_hbm.at[pl.ds(base + 16, ROWS_PER_W - 16)],
                  idx_v.at[pl.ds(16, ROWS_PER_W - 16)])

  def ensure(k):
    if 0 <= k < NCHUNK and k not in handles:
      handles[k] = start(k)

  ensure(1)
  for g in range(NCHUNK):
    ensure(g + 2)
    handles[g][0].wait()
    handles[g][1].wait()
    accs = accumulate(fbufs[g % NBUF], cbufs[g % NBUF], CHUNK_SIZES[g], accs)

  acc = accs[0]
  for a in accs[1:]:
    acc = acc + a
  res_v[...] = acc
  pltpu.sync_copy(res_v, out_hbm.at[wid])


@jax.jit
def _center_loss(features, labels, centers):
  labels2 = labels.astype(jnp.int32)
  mesh = plsc.VectorSubcoreMesh(core_axis_name="c", subcore_axis_name="s")
  run = pl.kernel(
      _body,
      out_type=jax.ShapeDtypeStruct((NW, L), jnp.float32),
      mesh=mesh,
      scratch_types=[
          pltpu.VMEM((ROWS_PER_W,), jnp.int32),
          pltpu.VMEM((CHUNK, FEATURE_DIM), jnp.float32),
          pltpu.VMEM((CHUNK, FEATURE_DIM), jnp.float32),
          pltpu.VMEM((CHUNK, FEATURE_DIM), jnp.float32),
          pltpu.VMEM((CHUNK, FEATURE_DIM), jnp.float32),
          pltpu.VMEM((CHUNK, FEATURE_DIM), jnp.float32),
          pltpu.VMEM((CHUNK, FEATURE_DIM), jnp.float32),
          pltpu.VMEM((L,), jnp.float32),
          pltpu.SemaphoreType.DMA,
          pltpu.SemaphoreType.DMA,
          pltpu.SemaphoreType.DMA,
          pltpu.SemaphoreType.DMA,
          pltpu.SemaphoreType.DMA,
          pltpu.SemaphoreType.DMA,
      ],
  )
  partials = run(features, labels2, centers)
  return jnp.sum(partials) / jnp.float32(BATCH * FEATURE_DIM)


def kernel(features, labels, centers):
  return _center_loss(features, labels, centers)
